# Initial kernel scaffold; baseline (speedup 1.0000x reference)
import functools

import jax
import jax.numpy as jnp
from jax import lax
from jax.experimental import pallas as pl
from jax.experimental.pallas import tpu as pltpu
from jax.experimental.pallas import tpu_sc as plsc

N_NODES = 10000
N_EDGES = 320000
D_FEAT = 128

NC = 2
NS = 16
NW = NC * NS
L = 16

CH = 128
EW = 10112
NCHUNK = EW // CH
E_PAD = EW * NW

N_PAD = 10240
RPT = N_PAD // NS
DUMP = 64
NDUMP = RPT // DUMP

_mesh = plsc.VectorSubcoreMesh(core_axis_name="c", subcore_axis_name="s")


@functools.partial(
    pl.kernel,
    out_type=jax.ShapeDtypeStruct((NC, N_PAD), jnp.float32),
    mesh=_mesh,
    scratch_types=[
        pltpu.VMEM((CH,), jnp.int32),
        pltpu.VMEM((N_PAD,), jnp.float32),
        pltpu.VMEM((RPT,), jnp.float32),
        pltpu.VMEM((RPT,), jnp.float32),
        pltpu.VMEM_SHARED((NS, N_PAD), jnp.float32),
    ],
)
def _deg_kernel(dst_hbm, out_hbm, dst_v, deg_v, acc_v, tmp_v, deg_sh):
    c = lax.axis_index("c")
    s = lax.axis_index("s")
    wid = s * NC + c
    zero = jnp.zeros((L,), jnp.float32)
    ones = jnp.ones((L,), jnp.float32)

    @pl.loop(0, N_PAD // L)
    def _zero(i):
        deg_v[pl.ds(i * L, L)] = zero

    @pl.loop(0, NCHUNK)
    def _count(j):
        off = wid * EW + j * CH
        pltpu.sync_copy(dst_hbm.at[pl.ds(off, CH)], dst_v)
        for i in range(CH // L):
            idx = dst_v[pl.ds(i * L, L)]
            plsc.addupdate_scatter(deg_v, [idx], ones)

    pltpu.sync_copy(deg_v, deg_sh.at[s])
    plsc.subcore_barrier()

    base = s * RPT

    @pl.loop(0, RPT // L)
    def _zero2(i):
        acc_v[pl.ds(i * L, L)] = zero

    for p in range(NS):
        pltpu.sync_copy(deg_sh.at[p, pl.ds(base, RPT)], tmp_v)

        @pl.loop(0, RPT // L)
        def _add(q):
            o = q * L
            acc_v[pl.ds(o, L)] = acc_v[pl.ds(o, L)] + tmp_v[pl.ds(o, L)]

    pltpu.sync_copy(acc_v, out_hbm.at[c, pl.ds(base, RPT)])


@functools.partial(
    pl.kernel,
    out_type=jax.ShapeDtypeStruct((NC, N_PAD, D_FEAT), jnp.float32),
    mesh=_mesh,
    scratch_types=[
        pltpu.VMEM((CH,), jnp.int32),
        pltpu.VMEM((CH,), jnp.int32),
        pltpu.VMEM((CH, D_FEAT), jnp.float32),
        pltpu.VMEM((DUMP, D_FEAT), jnp.float32),
        pltpu.VMEM_SHARED((N_PAD, D_FEAT), jnp.float32),
        pltpu.SemaphoreType.DMA,
    ],
)
def _agg_kernel(h_hbm, src_hbm, dst_hbm, out_hbm,
                src_v, dst_v, rows_v, buf_v, agg_sh, sem):
    c = lax.axis_index("c")
    s = lax.axis_index("s")
    wid = s * NC + c
    zero = jnp.zeros((L,), jnp.float32)

    @pl.loop(0, DUMP)
    def _zbuf(r):
        for q in range(D_FEAT // L):
            buf_v[r, pl.ds(q * L, L)] = zero

    base = s * RPT

    @pl.loop(0, NDUMP)
    def _zagg(j):
        pltpu.sync_copy(buf_v, agg_sh.at[pl.ds(base + j * DUMP, DUMP)])

    plsc.subcore_barrier()

    @pl.loop(0, NCHUNK)
    def _edges(j):
        off = wid * EW + j * CH
        pltpu.sync_copy(src_hbm.at[pl.ds(off, CH)], src_v)
        pltpu.sync_copy(dst_hbm.at[pl.ds(off, CH)], dst_v)
        pltpu.async_copy(h_hbm.at[src_v], rows_v, sem).wait()
        pltpu.sync_copy(rows_v, agg_sh.at[dst_v], add=True)

    plsc.subcore_barrier()

    @pl.loop(0, NDUMP)
    def _dump(j):
        r0 = base + j * DUMP
        pltpu.sync_copy(agg_sh.at[pl.ds(r0, DUMP)], buf_v)
        pltpu.sync_copy(buf_v, out_hbm.at[c, pl.ds(r0, DUMP)])


_RB = 1000


def _tc_init_body(f_ref, da_ref, db_ref, th_ref, h1_ref, hacc_ref):
    deg = jnp.maximum(da_ref[...] + db_ref[...], 1.0)
    dinv = 1.0 / jnp.sqrt(deg)
    f = f_ref[...]
    h1_ref[...] = f * dinv
    hacc_ref[...] = th_ref[0] * f


def _tc_round_body(k, f_ref, a0_ref, a1_ref, da_ref, db_ref, hacc_ref, th_ref,
                   fnew_ref, hout_ref, hnext_ref):
    deg = jnp.maximum(da_ref[...] + db_ref[...], 1.0)
    dinv = 1.0 / jnp.sqrt(deg)
    fnew = f_ref[...] - (a0_ref[...] + a1_ref[...]) * dinv
    fnew_ref[...] = fnew
    hout_ref[...] = hacc_ref[...] + th_ref[k] * fnew
    hnext_ref[...] = fnew * dinv


_row_spec = pl.BlockSpec((_RB, D_FEAT), lambda i: (i, 0))
_col_spec = pl.BlockSpec((_RB, 1), lambda i: (i, 0))
_smem_spec = pl.BlockSpec(memory_space=pltpu.MemorySpace.SMEM)
_fshape = jax.ShapeDtypeStruct((N_NODES, D_FEAT), jnp.float32)

_tc_init = pl.pallas_call(
    _tc_init_body,
    grid=(N_NODES // _RB,),
    in_specs=[_row_spec, _col_spec, _col_spec, _smem_spec],
    out_specs=[_row_spec, _row_spec],
    out_shape=[_fshape, _fshape],
)

_tc_round = [
    pl.pallas_call(
        functools.partial(_tc_round_body, k),
        grid=(N_NODES // _RB,),
        in_specs=[_row_spec, _row_spec, _row_spec, _col_spec, _col_spec,
                  _row_spec, _smem_spec],
        out_specs=[_row_spec, _row_spec, _row_spec],
        out_shape=[_fshape, _fshape, _fshape],
    )
    for k in range(4)
]


@jax.jit
def kernel(feat, edge_index, theta):
    pad = E_PAD - N_EDGES
    src = jnp.concatenate([edge_index[0], jnp.zeros((pad,), jnp.int32)])
    dst = jnp.concatenate(
        [edge_index[1], jnp.full((pad,), N_NODES, jnp.int32)])

    degp = _deg_kernel(dst)
    dega = degp[0, :N_NODES].reshape(N_NODES, 1)
    degb = degp[1, :N_NODES].reshape(N_NODES, 1)

    h, hacc = _tc_init(feat, dega, degb, theta)
    f = feat
    for k in range(1, 4):
        aggp = _agg_kernel(h, src, dst)
        f, hacc, h = _tc_round[k](
            f, aggp[0, :N_NODES], aggp[1, :N_NODES], dega, degb, hacc, theta)
    return hacc


# speedup vs baseline: 3.2730x; 3.2730x over previous
import functools

import jax
import jax.numpy as jnp
from jax import lax
from jax.experimental import pallas as pl
from jax.experimental.pallas import tpu as pltpu
from jax.experimental.pallas import tpu_sc as plsc

N_NODES = 10000
N_EDGES = 320000
D_FEAT = 128

NC = 2
NS = 16
NW = NC * NS
L = 16

CH = 128
EW = 10112
NCHUNK = EW // CH
E_PAD = EW * NW

N_PAD = 10240
RPT = N_PAD // NS
DUMP = 64
NDUMP = RPT // DUMP

_mesh = plsc.VectorSubcoreMesh(core_axis_name="c", subcore_axis_name="s")


@functools.partial(
    pl.kernel,
    out_type=jax.ShapeDtypeStruct((NC, N_PAD), jnp.float32),
    mesh=_mesh,
    scratch_types=[
        pltpu.VMEM((CH,), jnp.int32),
        pltpu.VMEM((N_PAD,), jnp.float32),
        pltpu.VMEM((RPT,), jnp.float32),
        pltpu.VMEM((RPT,), jnp.float32),
        pltpu.VMEM_SHARED((NS, N_PAD), jnp.float32),
    ],
    compiler_params=pltpu.CompilerParams(needs_layout_passes=False),
)
def _deg_kernel(dst_hbm, out_hbm, dst_v, deg_v, acc_v, tmp_v, deg_sh):
    c = lax.axis_index("c")
    s = lax.axis_index("s")
    wid = s * NC + c
    zero = jnp.zeros((L,), jnp.float32)
    ones = jnp.ones((L,), jnp.float32)

    @pl.loop(0, N_PAD // L)
    def _zero(i):
        deg_v[pl.ds(i * L, L)] = zero

    @pl.loop(0, NCHUNK)
    def _count(j):
        off = wid * EW + j * CH
        pltpu.sync_copy(dst_hbm.at[pl.ds(off, CH)], dst_v)
        for i in range(CH // L):
            idx = dst_v[pl.ds(i * L, L)]
            plsc.addupdate_scatter(deg_v, [idx], ones)

    pltpu.sync_copy(deg_v, deg_sh.at[s])
    plsc.subcore_barrier()

    base = s * RPT

    @pl.loop(0, RPT // L)
    def _zero2(i):
        acc_v[pl.ds(i * L, L)] = zero

    for p in range(NS):
        pltpu.sync_copy(deg_sh.at[p, pl.ds(base, RPT)], tmp_v)

        @pl.loop(0, RPT // L)
        def _add(q):
            o = q * L
            acc_v[pl.ds(o, L)] = acc_v[pl.ds(o, L)] + tmp_v[pl.ds(o, L)]

    pltpu.sync_copy(acc_v, out_hbm.at[c, pl.ds(base, RPT)])


@functools.partial(
    pl.kernel,
    out_type=jax.ShapeDtypeStruct((NC, N_PAD, D_FEAT), jnp.float32),
    mesh=_mesh,
    scratch_types=[
        pltpu.VMEM((CH,), jnp.int32),
        pltpu.VMEM((CH,), jnp.int32),
        pltpu.VMEM((CH, D_FEAT), jnp.float32),
        pltpu.VMEM((DUMP, D_FEAT), jnp.float32),
        pltpu.VMEM_SHARED((N_PAD, D_FEAT), jnp.float32),
        pltpu.SemaphoreType.DMA,
    ],
    compiler_params=pltpu.CompilerParams(needs_layout_passes=False),
)
def _agg_kernel(h_hbm, src_hbm, dst_hbm, out_hbm,
                src_v, dst_v, rows_v, buf_v, agg_sh, sem):
    c = lax.axis_index("c")
    s = lax.axis_index("s")
    wid = s * NC + c
    zero = jnp.zeros((L,), jnp.float32)

    @pl.loop(0, DUMP)
    def _zbuf(r):
        for q in range(D_FEAT // L):
            buf_v[r, pl.ds(q * L, L)] = zero

    base = s * RPT

    @pl.loop(0, NDUMP)
    def _zagg(j):
        pltpu.sync_copy(buf_v, agg_sh.at[pl.ds(base + j * DUMP, DUMP)])

    plsc.subcore_barrier()

    @pl.loop(0, NCHUNK)
    def _edges(j):
        off = wid * EW + j * CH
        pltpu.sync_copy(src_hbm.at[pl.ds(off, CH)], src_v)
        pltpu.sync_copy(dst_hbm.at[pl.ds(off, CH)], dst_v)
        pltpu.async_copy(h_hbm.at[src_v], rows_v, sem).wait()
        pltpu.sync_copy(rows_v, agg_sh.at[dst_v], add=True)

    plsc.subcore_barrier()

    @pl.loop(0, NDUMP)
    def _dump(j):
        r0 = base + j * DUMP
        pltpu.sync_copy(agg_sh.at[pl.ds(r0, DUMP)], buf_v)
        pltpu.sync_copy(buf_v, out_hbm.at[c, pl.ds(r0, DUMP)])


_RB = 1000


def _tc_init_body(f_ref, da_ref, db_ref, th_ref, h1_ref, hacc_ref):
    deg = jnp.maximum(da_ref[...] + db_ref[...], 1.0)
    dinv = 1.0 / jnp.sqrt(deg)
    f = f_ref[...]
    h1_ref[...] = f * dinv
    hacc_ref[...] = th_ref[0] * f


def _tc_round_body(k, f_ref, a0_ref, a1_ref, da_ref, db_ref, hacc_ref, th_ref,
                   fnew_ref, hout_ref, hnext_ref):
    deg = jnp.maximum(da_ref[...] + db_ref[...], 1.0)
    dinv = 1.0 / jnp.sqrt(deg)
    fnew = f_ref[...] - (a0_ref[...] + a1_ref[...]) * dinv
    fnew_ref[...] = fnew
    hout_ref[...] = hacc_ref[...] + th_ref[k] * fnew
    hnext_ref[...] = fnew * dinv


_row_spec = pl.BlockSpec((_RB, D_FEAT), lambda i: (i, 0))
_col_spec = pl.BlockSpec((_RB, 1), lambda i: (i, 0))
_smem_spec = pl.BlockSpec(memory_space=pltpu.MemorySpace.SMEM)
_fshape = jax.ShapeDtypeStruct((N_NODES, D_FEAT), jnp.float32)

_tc_init = pl.pallas_call(
    _tc_init_body,
    grid=(N_NODES // _RB,),
    in_specs=[_row_spec, _col_spec, _col_spec, _smem_spec],
    out_specs=[_row_spec, _row_spec],
    out_shape=[_fshape, _fshape],
)

_tc_round = [
    pl.pallas_call(
        functools.partial(_tc_round_body, k),
        grid=(N_NODES // _RB,),
        in_specs=[_row_spec, _row_spec, _row_spec, _col_spec, _col_spec,
                  _row_spec, _smem_spec],
        out_specs=[_row_spec, _row_spec, _row_spec],
        out_shape=[_fshape, _fshape, _fshape],
    )
    for k in range(4)
]


@jax.jit
def kernel(feat, edge_index, theta):
    pad = E_PAD - N_EDGES
    src = jnp.concatenate([edge_index[0], jnp.zeros((pad,), jnp.int32)])
    dst = jnp.concatenate(
        [edge_index[1], jnp.full((pad,), N_NODES, jnp.int32)])

    degp = _deg_kernel(dst)
    dega = degp[0, :N_NODES].reshape(N_NODES, 1)
    degb = degp[1, :N_NODES].reshape(N_NODES, 1)

    h, hacc = _tc_init(feat, dega, degb, theta)
    f = feat
    for k in range(1, 4):
        aggp = _agg_kernel(h, src, dst)
        f, hacc, h = _tc_round[k](
            f, aggp[0, :N_NODES], aggp[1, :N_NODES], dega, degb, hacc, theta)
    return hacc
